# Initial kernel scaffold; baseline (speedup 1.0000x reference)
import jax
import jax.numpy as jnp
from jax.experimental import pallas as pl


def kernel(vector_tensor, edge_index_tensor, article1_idx, article2_idx, W1_self, W1_neigh, b1, W2_self, W2_neigh, b2, Wc, bc):
    raise NotImplementedError("write your pallas kernel here")


# speedup vs baseline: 12.0262x; 12.0262x over previous
import functools

import jax
import jax.numpy as jnp
from jax import lax
from jax.experimental import pallas as pl
from jax.experimental.pallas import tpu as pltpu
from jax.experimental.pallas import tpu_sc as plsc

NC = 2
NS = 16
NW = NC * NS
L = 16


def _mm2_body(x_ref, ws_ref, wn_ref, z_ref, m_ref):
  x = x_ref[...]
  z_ref[...] = jnp.dot(x, ws_ref[...], preferred_element_type=jnp.float32)
  m_ref[...] = jnp.dot(x, wn_ref[...], preferred_element_type=jnp.float32)


def _x1_body(z_ref, s_ref, d_ref, b_ref, o_ref):
  n = o_ref.shape[0]
  deg = jnp.maximum(d_ref[0, :n] + d_ref[1, :n], 1.0)
  agg = (s_ref[0, :n] + s_ref[1, :n]) / deg
  o_ref[...] = jnp.maximum(z_ref[...] + agg + b_ref[...], 0.0)


def _uv_body(x1_ref, s_ref, d_ref, ws_ref, wn_ref, b2_ref, wc_ref, bc_ref,
             o_ref):
  n = o_ref.shape[0]
  deg = jnp.maximum(d_ref[0, :n] + d_ref[1, :n], 1.0)
  agg = (s_ref[0, :n] + s_ref[1, :n]) / deg
  wct = wc_ref[...].reshape(2, 128).T
  ps = jnp.dot(ws_ref[...], wct, preferred_element_type=jnp.float32)
  pn = jnp.dot(wn_ref[...], wct, preferred_element_type=jnp.float32)
  base = jnp.dot(b2_ref[...], wct, preferred_element_type=jnp.float32)
  base = base + jnp.concatenate([bc_ref[...], jnp.zeros((1, 1), jnp.float32)],
                                axis=1)
  o_ref[...] = (jnp.dot(x1_ref[...], ps, preferred_element_type=jnp.float32)
                + jnp.dot(agg, pn, preferred_element_type=jnp.float32)
                + base)


def _edge_pass(src3, dst3, table, n, with_deg):
  ch_r, ch_c = src3.shape[1], src3.shape[2]
  n_pad = ((n + 8 * NS - 1) // (8 * NS)) * (8 * NS)
  stripe = n_pad // NS
  mesh = plsc.VectorSubcoreMesh(core_axis_name="c", subcore_axis_name="s")

  out_type = [jax.ShapeDtypeStruct((NC, n_pad, L), jnp.float32)]
  scratch = [
      pltpu.VMEM((ch_r, ch_c), jnp.int32),
      pltpu.VMEM((ch_r, ch_c), jnp.int32),
      pltpu.VMEM((ch_c, L), jnp.float32),
      pltpu.VMEM((stripe, L), jnp.float32),
      pltpu.VMEM_SHARED((n_pad, L), jnp.float32),
  ]
  if with_deg:
    out_type.append(jax.ShapeDtypeStruct((NC, n_pad, L), jnp.float32))
    scratch.append(pltpu.VMEM((ch_c, L), jnp.float32))
    scratch.append(pltpu.VMEM_SHARED((n_pad, L), jnp.float32))

  def body(src_h, dst_h, tab_h, *rest):
    if with_deg:
      (sum_h, deg_h, src_v, dst_v, rows_v, stripe_v, acc_s, ones_v,
       dacc_s) = rest
    else:
      sum_h, src_v, dst_v, rows_v, stripe_v, acc_s = rest
    cid = lax.axis_index("c")
    sid = lax.axis_index("s")
    wid = sid * NC + cid

    def zrow(i, carry):
      stripe_v[i] = jnp.zeros((L,), jnp.float32)
      return carry
    lax.fori_loop(0, stripe, zrow, 0)
    pltpu.sync_copy(stripe_v, acc_s.at[pl.ds(sid * stripe, stripe)])
    if with_deg:
      pltpu.sync_copy(stripe_v, dacc_s.at[pl.ds(sid * stripe, stripe)])

      def orow(i, carry):
        ones_v[i] = jnp.ones((L,), jnp.float32)
        return carry
      lax.fori_loop(0, ch_c, orow, 0)

    pltpu.sync_copy(src_h.at[wid], src_v)
    pltpu.sync_copy(dst_h.at[wid], dst_v)
    plsc.subcore_barrier()

    def step(j, carry):
      pltpu.sync_copy(tab_h.at[src_v.at[j]], rows_v)
      pltpu.sync_copy(rows_v, acc_s.at[dst_v.at[j]], add=True)
      if with_deg:
        pltpu.sync_copy(ones_v, dacc_s.at[dst_v.at[j]], add=True)
      return carry
    lax.fori_loop(0, ch_r, step, 0)

    plsc.subcore_barrier()
    sl = pl.ds(sid * stripe, stripe)
    pltpu.sync_copy(acc_s.at[sl], stripe_v)
    pltpu.sync_copy(stripe_v, sum_h.at[cid, sl])
    if with_deg:
      pltpu.sync_copy(dacc_s.at[sl], stripe_v)
      pltpu.sync_copy(stripe_v, deg_h.at[cid, sl])

  run = pl.kernel(body, out_type=out_type, mesh=mesh, scratch_types=scratch,
                  compiler_params=pltpu.CompilerParams(
                      use_tc_tiling_on_sc=False))
  return run(src3, dst3, table)


def _pair_gather(uv, a1, a2, b):
  n = uv.shape[0]
  bw = b // NW
  mesh = plsc.VectorSubcoreMesh(core_axis_name="c", subcore_axis_name="s")

  def body(uv_h, a1_h, a2_h, out_h, uv_v, a1_v, a2_v, res_v):
    cid = lax.axis_index("c")
    sid = lax.axis_index("s")
    wid = sid * NC + cid
    base = wid * bw
    pltpu.sync_copy(uv_h, uv_v)
    pltpu.sync_copy(a1_h.at[pl.ds(base, bw)], a1_v)
    pltpu.sync_copy(a2_h.at[pl.ds(base, bw)], a2_v)
    col0 = jnp.zeros((L,), jnp.int32)
    col1 = col0 + 1

    def step(j, carry):
      i1 = a1_v[pl.ds(j * L, L)]
      i2 = a2_v[pl.ds(j * L, L)]
      g1 = plsc.load_gather(uv_v, [i1, col0])
      g2 = plsc.load_gather(uv_v, [i2, col1])
      res_v[pl.ds(j * L, L)] = g1 + g2
      return carry
    lax.fori_loop(0, bw // L, step, 0)
    pltpu.sync_copy(res_v, out_h.at[pl.ds(base, bw)])

  run = pl.kernel(
      body,
      out_type=[jax.ShapeDtypeStruct((b,), jnp.float32)],
      mesh=mesh,
      scratch_types=[
          pltpu.VMEM((n, 2), jnp.float32),
          pltpu.VMEM((bw,), jnp.int32),
          pltpu.VMEM((bw,), jnp.int32),
          pltpu.VMEM((bw,), jnp.float32),
      ],
      compiler_params=pltpu.CompilerParams(use_tc_tiling_on_sc=False,
                                           needs_layout_passes=False),
  )
  (out,) = run(uv, a1, a2)
  return out


def kernel(vector_tensor, edge_index_tensor, article1_idx, article2_idx,
           W1_self, W1_neigh, b1, W2_self, W2_neigh, b2, Wc, bc):
  n, d = vector_tensor.shape
  e = edge_index_tensor.shape[1]
  h = W1_self.shape[1]
  b = article1_idx.shape[0]
  ew = e // NW
  ch_c = 100
  ch_r = ew // ch_c

  src3 = edge_index_tensor[0].astype(jnp.int32).reshape(NW, ch_r, ch_c)
  dst3 = edge_index_tensor[1].astype(jnp.int32).reshape(NW, ch_r, ch_c)
  a1 = article1_idx.astype(jnp.int32)
  a2 = article2_idx.astype(jnp.int32)

  z, m1 = pl.pallas_call(
      _mm2_body,
      out_shape=[jax.ShapeDtypeStruct((n, h), jnp.float32),
                 jax.ShapeDtypeStruct((n, h), jnp.float32)],
  )(vector_tensor, W1_self, W1_neigh)

  s1, degp = _edge_pass(src3, dst3, m1, n, with_deg=True)

  x1 = pl.pallas_call(
      _x1_body,
      out_shape=jax.ShapeDtypeStruct((n, h), jnp.float32),
  )(z, s1, degp, b1.reshape(1, h))

  (s2,) = _edge_pass(src3, dst3, x1, n, with_deg=False)

  uv = pl.pallas_call(
      _uv_body,
      out_shape=jax.ShapeDtypeStruct((n, 2), jnp.float32),
  )(x1, s2, degp, W2_self, W2_neigh, b2.reshape(1, d), Wc,
    bc.reshape(1, 1))

  logits = _pair_gather(uv, a1, a2, b)
  return logits.reshape(b, 1)


# speedup vs baseline: 22.0445x; 1.8330x over previous
import functools

import jax
import jax.numpy as jnp
from jax import lax
from jax.experimental import pallas as pl
from jax.experimental.pallas import tpu as pltpu
from jax.experimental.pallas import tpu_sc as plsc

NC = 2
NS = 16
NW = NC * NS
L = 16


def _mm2_body(x_ref, ws_ref, wn_ref, z_ref, m_ref):
  x = x_ref[...]
  z_ref[...] = jnp.dot(x, ws_ref[...], preferred_element_type=jnp.float32)
  m_ref[...] = jnp.dot(x, wn_ref[...], preferred_element_type=jnp.float32)


def _x1_body(z_ref, s_ref, d_ref, b_ref, o_ref):
  n = o_ref.shape[0]
  deg = jnp.maximum(d_ref[0, :n] + d_ref[1, :n], 1.0)
  agg = (s_ref[0, :n] + s_ref[1, :n]) / deg
  o_ref[...] = jnp.maximum(z_ref[...] + agg + b_ref[...], 0.0)


def _uv_body(x1_ref, s_ref, d_ref, ws_ref, wn_ref, b2_ref, wc_ref, bc_ref,
             o_ref):
  n = o_ref.shape[0]
  deg = jnp.maximum(d_ref[0, :n] + d_ref[1, :n], 1.0)
  agg = (s_ref[0, :n] + s_ref[1, :n]) / deg
  wct = wc_ref[...].reshape(2, 128).T
  ps = jnp.dot(ws_ref[...], wct, preferred_element_type=jnp.float32)
  pn = jnp.dot(wn_ref[...], wct, preferred_element_type=jnp.float32)
  base = jnp.dot(b2_ref[...], wct, preferred_element_type=jnp.float32)
  base = base + jnp.concatenate([bc_ref[...], jnp.zeros((1, 1), jnp.float32)],
                                axis=1)
  o_ref[...] = (jnp.dot(x1_ref[...], ps, preferred_element_type=jnp.float32)
                + jnp.dot(agg, pn, preferred_element_type=jnp.float32)
                + base)


def _edge_pass(src3, dst3, table, n, with_deg):
  ch_r, ch_c = src3.shape[1], src3.shape[2]
  n_pad = ((n + 8 * NS - 1) // (8 * NS)) * (8 * NS)
  stripe = n_pad // NS
  mesh = plsc.VectorSubcoreMesh(core_axis_name="c", subcore_axis_name="s")

  nb = 10
  lag = 5
  assert ch_r % nb == 0 and ch_r >= nb

  out_type = [jax.ShapeDtypeStruct((NC, n_pad, L), jnp.float32)]
  scratch = [
      pltpu.VMEM((ch_r, ch_c), jnp.int32),
      pltpu.VMEM((ch_r, ch_c), jnp.int32),
      pltpu.VMEM((nb, ch_c, L), jnp.float32),
      pltpu.VMEM((stripe, L), jnp.float32),
      pltpu.VMEM_SHARED((n_pad, L), jnp.float32),
      pltpu.SemaphoreType.DMA((nb,)),
      pltpu.SemaphoreType.DMA((nb,)),
  ]
  if with_deg:
    out_type.append(jax.ShapeDtypeStruct((NC, n_pad, L), jnp.float32))
    scratch.append(pltpu.VMEM((ch_c, L), jnp.float32))
    scratch.append(pltpu.VMEM_SHARED((n_pad, L), jnp.float32))
    scratch.append(pltpu.SemaphoreType.DMA((nb,)))

  def body(src_h, dst_h, tab_h, *rest):
    if with_deg:
      (sum_h, deg_h, src_v, dst_v, rows_v, stripe_v, acc_s, gsem, ssem,
       ones_v, dacc_s, osem) = rest
    else:
      sum_h, src_v, dst_v, rows_v, stripe_v, acc_s, gsem, ssem = rest
    cid = lax.axis_index("c")
    sid = lax.axis_index("s")
    wid = sid * NC + cid

    def zrow(i, carry):
      stripe_v[i] = jnp.zeros((L,), jnp.float32)
      return carry
    lax.fori_loop(0, stripe, zrow, 0)
    pltpu.sync_copy(stripe_v, acc_s.at[pl.ds(sid * stripe, stripe)])
    if with_deg:
      pltpu.sync_copy(stripe_v, dacc_s.at[pl.ds(sid * stripe, stripe)])

      def orow(i, carry):
        ones_v[i] = jnp.ones((L,), jnp.float32)
        return carry
      lax.fori_loop(0, ch_c, orow, 0)

    pltpu.sync_copy(src_h.at[wid], src_v)
    pltpu.sync_copy(dst_h.at[wid], dst_v)

    for b in range(nb):
      pltpu.async_copy(tab_h.at[src_v.at[b]], rows_v.at[b], gsem.at[b])
    plsc.subcore_barrier()

    def group(g, carry):
      for b in range(nb):
        j = g * nb + b
        pltpu.make_async_copy(tab_h.at[src_v.at[j]], rows_v.at[b],
                              gsem.at[b]).wait()
        pltpu.async_copy(rows_v.at[b], acc_s.at[dst_v.at[j]], ssem.at[b],
                         add=True)
        if with_deg:
          pltpu.async_copy(ones_v, dacc_s.at[dst_v.at[j]], osem.at[b],
                           add=True)
        sd = j - lag
        bd = (b - lag) % nb

        @pl.when(sd >= 0)
        def _():
          pltpu.make_async_copy(rows_v.at[bd], acc_s.at[dst_v.at[0]],
                                ssem.at[bd]).wait()
          if with_deg:
            pltpu.make_async_copy(ones_v, dacc_s.at[dst_v.at[0]],
                                  osem.at[bd]).wait()
          jg = sd + nb

          @pl.when(jg < ch_r)
          def _():
            pltpu.async_copy(tab_h.at[src_v.at[jg]], rows_v.at[bd],
                             gsem.at[bd])
      return carry
    lax.fori_loop(0, ch_r // nb, group, 0)

    for t in range(lag):
      bd = (ch_r - lag + t) % nb
      pltpu.make_async_copy(rows_v.at[bd], acc_s.at[dst_v.at[0]],
                            ssem.at[bd]).wait()
      if with_deg:
        pltpu.make_async_copy(ones_v, dacc_s.at[dst_v.at[0]],
                              osem.at[bd]).wait()

    plsc.subcore_barrier()
    sl = pl.ds(sid * stripe, stripe)
    pltpu.sync_copy(acc_s.at[sl], stripe_v)
    pltpu.sync_copy(stripe_v, sum_h.at[cid, sl])
    if with_deg:
      pltpu.sync_copy(dacc_s.at[sl], stripe_v)
      pltpu.sync_copy(stripe_v, deg_h.at[cid, sl])

  run = pl.kernel(body, out_type=out_type, mesh=mesh, scratch_types=scratch,
                  compiler_params=pltpu.CompilerParams(
                      use_tc_tiling_on_sc=False))
  return run(src3, dst3, table)


def _pair_gather(uv, a1, a2, b):
  n = uv.shape[0]
  bw = b // NW
  mesh = plsc.VectorSubcoreMesh(core_axis_name="c", subcore_axis_name="s")

  def body(uv_h, a1_h, a2_h, out_h, uv_v, a1_v, a2_v, res_v):
    cid = lax.axis_index("c")
    sid = lax.axis_index("s")
    wid = sid * NC + cid
    base = wid * bw
    pltpu.sync_copy(uv_h, uv_v)
    pltpu.sync_copy(a1_h.at[pl.ds(base, bw)], a1_v)
    pltpu.sync_copy(a2_h.at[pl.ds(base, bw)], a2_v)
    col0 = jnp.zeros((L,), jnp.int32)
    col1 = col0 + 1

    def step(j, carry):
      i1 = a1_v[pl.ds(j * L, L)]
      i2 = a2_v[pl.ds(j * L, L)]
      g1 = plsc.load_gather(uv_v, [i1, col0])
      g2 = plsc.load_gather(uv_v, [i2, col1])
      res_v[pl.ds(j * L, L)] = g1 + g2
      return carry
    lax.fori_loop(0, bw // L, step, 0)
    pltpu.sync_copy(res_v, out_h.at[pl.ds(base, bw)])

  run = pl.kernel(
      body,
      out_type=[jax.ShapeDtypeStruct((b,), jnp.float32)],
      mesh=mesh,
      scratch_types=[
          pltpu.VMEM((n, 2), jnp.float32),
          pltpu.VMEM((bw,), jnp.int32),
          pltpu.VMEM((bw,), jnp.int32),
          pltpu.VMEM((bw,), jnp.float32),
      ],
      compiler_params=pltpu.CompilerParams(use_tc_tiling_on_sc=False,
                                           needs_layout_passes=False),
  )
  (out,) = run(uv, a1, a2)
  return out


def kernel(vector_tensor, edge_index_tensor, article1_idx, article2_idx,
           W1_self, W1_neigh, b1, W2_self, W2_neigh, b2, Wc, bc):
  n, d = vector_tensor.shape
  e = edge_index_tensor.shape[1]
  h = W1_self.shape[1]
  b = article1_idx.shape[0]
  ew = e // NW
  ch_c = 100
  ch_r = ew // ch_c

  src3 = edge_index_tensor[0].astype(jnp.int32).reshape(NW, ch_r, ch_c)
  dst3 = edge_index_tensor[1].astype(jnp.int32).reshape(NW, ch_r, ch_c)
  a1 = article1_idx.astype(jnp.int32)
  a2 = article2_idx.astype(jnp.int32)

  z, m1 = pl.pallas_call(
      _mm2_body,
      out_shape=[jax.ShapeDtypeStruct((n, h), jnp.float32),
                 jax.ShapeDtypeStruct((n, h), jnp.float32)],
  )(vector_tensor, W1_self, W1_neigh)

  s1, degp = _edge_pass(src3, dst3, m1, n, with_deg=True)

  x1 = pl.pallas_call(
      _x1_body,
      out_shape=jax.ShapeDtypeStruct((n, h), jnp.float32),
  )(z, s1, degp, b1.reshape(1, h))

  (s2,) = _edge_pass(src3, dst3, x1, n, with_deg=False)

  uv = pl.pallas_call(
      _uv_body,
      out_shape=jax.ShapeDtypeStruct((n, 2), jnp.float32),
  )(x1, s2, degp, W2_self, W2_neigh, b2.reshape(1, d), Wc,
    bc.reshape(1, 1))

  logits = _pair_gather(uv, a1, a2, b)
  return logits.reshape(b, 1)


# speedup vs baseline: 25.6634x; 1.1642x over previous
import functools

import jax
import jax.numpy as jnp
from jax import lax
from jax.experimental import pallas as pl
from jax.experimental.pallas import tpu as pltpu
from jax.experimental.pallas import tpu_sc as plsc

NC = 2
NS = 16
NW = NC * NS
L = 16


def _mm2_body(x_ref, ws_ref, wn_ref, z_ref, m_ref):
  x = x_ref[...]
  n = x.shape[0]
  z_ref[0:n] = jnp.dot(x, ws_ref[...], preferred_element_type=jnp.float32)
  z_ref[n:] = jnp.zeros((z_ref.shape[0] - n, z_ref.shape[1]), jnp.float32)
  m_ref[...] = jnp.dot(x, wn_ref[...], preferred_element_type=jnp.float32)


def _uv_body(x1_ref, s_ref, d_ref, ws_ref, wn_ref, b2_ref, wc_ref, bc_ref,
             o_ref):
  n = o_ref.shape[0]
  x1 = x1_ref[0:n]
  deg = jnp.maximum(d_ref[0, :n] + d_ref[1, :n], 1.0)
  agg = (s_ref[0, :n] + s_ref[1, :n]) / deg
  wct = wc_ref[...].reshape(2, 128).T
  ps = jnp.dot(ws_ref[...], wct, preferred_element_type=jnp.float32)
  pn = jnp.dot(wn_ref[...], wct, preferred_element_type=jnp.float32)
  base = jnp.dot(b2_ref[...], wct, preferred_element_type=jnp.float32)
  base = base + jnp.concatenate([bc_ref[...], jnp.zeros((1, 1), jnp.float32)],
                                axis=1)
  o_ref[...] = (jnp.dot(x1, ps, preferred_element_type=jnp.float32)
                + jnp.dot(agg, pn, preferred_element_type=jnp.float32)
                + base)


def _edge_pass(ei3, n_pad, table=None, x1_in=None, with_deg=False):
  ch_r, ch_c = ei3.shape[2], ei3.shape[3]
  stripe = n_pad // NS
  mesh = plsc.VectorSubcoreMesh(core_axis_name="c", subcore_axis_name="s")

  nb = 10
  lag = 5
  assert ch_r % nb == 0 and ch_r >= nb

  out_type = [jax.ShapeDtypeStruct((NC, n_pad, L), jnp.float32)]
  scratch = [
      pltpu.VMEM((ch_r, ch_c), jnp.int32),
      pltpu.VMEM((ch_r, ch_c), jnp.int32),
      pltpu.VMEM((nb, ch_c, L), jnp.float32),
      pltpu.VMEM((stripe, L), jnp.float32),
      pltpu.VMEM_SHARED((n_pad, L), jnp.float32),
      pltpu.SemaphoreType.DMA((nb,)),
      pltpu.SemaphoreType.DMA((nb,)),
  ]
  if with_deg:
    out_type.append(jax.ShapeDtypeStruct((NC, n_pad, L), jnp.float32))
    scratch.append(pltpu.VMEM((ch_c, L), jnp.float32))
    scratch.append(pltpu.VMEM_SHARED((n_pad, L), jnp.float32))
    scratch.append(pltpu.SemaphoreType.DMA((nb,)))
  if x1_in is not None:
    out_type.append(jax.ShapeDtypeStruct((n_pad, L), jnp.float32))
    scratch.append(pltpu.VMEM((stripe, L), jnp.float32))
    scratch.append(pltpu.VMEM((stripe, L), jnp.float32))
    scratch.append(pltpu.VMEM((stripe, L), jnp.float32))
    scratch.append(pltpu.VMEM((stripe, L), jnp.float32))
    scratch.append(pltpu.VMEM((L,), jnp.float32))
    scratch.append(pltpu.VMEM_SHARED((n_pad, L), jnp.float32))

  def body(*refs):
    it = iter(refs)
    ei_h = next(it)
    if x1_in is not None:
      z_h, s1_h, dg_h, b1_h = next(it), next(it), next(it), next(it)
    else:
      tab_h = next(it)
    sum_h = next(it)
    if with_deg:
      deg_h = next(it)
    if x1_in is not None:
      x1_h = next(it)
    src_v, dst_v, rows_v, stripe_v, acc_s, gsem, ssem = (
        next(it), next(it), next(it), next(it), next(it), next(it), next(it))
    if with_deg:
      ones_v, dacc_s, osem = next(it), next(it), next(it)
    if x1_in is not None:
      abuf, bbuf, dabuf, dbbuf, b1v, tab_s = (
          next(it), next(it), next(it), next(it), next(it), next(it))
    cid = lax.axis_index("c")
    sid = lax.axis_index("s")
    wid = sid * NC + cid
    sl = pl.ds(sid * stripe, stripe)

    def zrow(i, carry):
      stripe_v[i] = jnp.zeros((L,), jnp.float32)
      return carry
    lax.fori_loop(0, stripe, zrow, 0)
    pltpu.sync_copy(stripe_v, acc_s.at[sl])
    if with_deg:
      pltpu.sync_copy(stripe_v, dacc_s.at[sl])

      def orow(i, carry):
        ones_v[i] = jnp.ones((L,), jnp.float32)
        return carry
      lax.fori_loop(0, ch_c, orow, 0)

    pltpu.sync_copy(ei_h.at[0, wid], src_v)
    pltpu.sync_copy(ei_h.at[1, wid], dst_v)

    if x1_in is not None:
      pltpu.sync_copy(z_h.at[sl], stripe_v)
      pltpu.sync_copy(s1_h.at[0, sl], abuf)
      pltpu.sync_copy(s1_h.at[1, sl], bbuf)
      pltpu.sync_copy(dg_h.at[0, sl], dabuf)
      pltpu.sync_copy(dg_h.at[1, sl], dbbuf)
      pltpu.sync_copy(b1_h, b1v)
      b1vec = b1v[...]

      def xrow(i, carry):
        s = abuf[i] + bbuf[i]
        d = jnp.maximum(dabuf[i] + dbbuf[i], 1.0)
        stripe_v[i] = jnp.maximum(stripe_v[i] + s / d + b1vec, 0.0)
        return carry
      lax.fori_loop(0, stripe, xrow, 0)
      pltpu.sync_copy(stripe_v, tab_s.at[sl])

      @pl.when(cid == 0)
      def _():
        pltpu.sync_copy(stripe_v, x1_h.at[sl])
      plsc.subcore_barrier()
      tab = tab_s
    else:
      tab = tab_h

    for b in range(nb):
      pltpu.async_copy(tab.at[src_v.at[b]], rows_v.at[b], gsem.at[b])
    if x1_in is None:
      plsc.subcore_barrier()

    def group(g, carry):
      for b in range(nb):
        j = g * nb + b
        pltpu.make_async_copy(tab.at[src_v.at[j]], rows_v.at[b],
                              gsem.at[b]).wait()
        pltpu.async_copy(rows_v.at[b], acc_s.at[dst_v.at[j]], ssem.at[b],
                         add=True)
        if with_deg:
          pltpu.async_copy(ones_v, dacc_s.at[dst_v.at[j]], osem.at[b],
                           add=True)
        sd = j - lag
        bd = (b - lag) % nb

        @pl.when(sd >= 0)
        def _():
          pltpu.make_async_copy(rows_v.at[bd], acc_s.at[dst_v.at[0]],
                                ssem.at[bd]).wait()
          if with_deg:
            pltpu.make_async_copy(ones_v, dacc_s.at[dst_v.at[0]],
                                  osem.at[bd]).wait()
          jg = sd + nb

          @pl.when(jg < ch_r)
          def _():
            pltpu.async_copy(tab.at[src_v.at[jg]], rows_v.at[bd],
                             gsem.at[bd])
      return carry
    lax.fori_loop(0, ch_r // nb, group, 0)

    for t in range(lag):
      bd = (ch_r - lag + t) % nb
      pltpu.make_async_copy(rows_v.at[bd], acc_s.at[dst_v.at[0]],
                            ssem.at[bd]).wait()
      if with_deg:
        pltpu.make_async_copy(ones_v, dacc_s.at[dst_v.at[0]],
                              osem.at[bd]).wait()

    plsc.subcore_barrier()
    pltpu.sync_copy(acc_s.at[sl], stripe_v)
    pltpu.sync_copy(stripe_v, sum_h.at[cid, sl])
    if with_deg:
      pltpu.sync_copy(dacc_s.at[sl], stripe_v)
      pltpu.sync_copy(stripe_v, deg_h.at[cid, sl])

  run = pl.kernel(body, out_type=out_type, mesh=mesh, scratch_types=scratch,
                  compiler_params=pltpu.CompilerParams(
                      use_tc_tiling_on_sc=False))
  if x1_in is not None:
    return run(ei3, *x1_in)
  return run(ei3, table)


def _pair_gather(uv, a1, a2, b):
  n = uv.shape[0]
  bw = b // NW
  mesh = plsc.VectorSubcoreMesh(core_axis_name="c", subcore_axis_name="s")

  def body(uv_h, a1_h, a2_h, out_h, uv_v, a1_v, a2_v, res_v):
    cid = lax.axis_index("c")
    sid = lax.axis_index("s")
    wid = sid * NC + cid
    base = wid * bw
    pltpu.sync_copy(uv_h, uv_v)
    pltpu.sync_copy(a1_h.at[pl.ds(base, bw)], a1_v)
    pltpu.sync_copy(a2_h.at[pl.ds(base, bw)], a2_v)
    col0 = jnp.zeros((L,), jnp.int32)
    col1 = col0 + 1

    def step(j, carry):
      i1 = a1_v[pl.ds(j * L, L)]
      i2 = a2_v[pl.ds(j * L, L)]
      g1 = plsc.load_gather(uv_v, [i1, col0])
      g2 = plsc.load_gather(uv_v, [i2, col1])
      res_v[pl.ds(j * L, L)] = g1 + g2
      return carry
    lax.fori_loop(0, bw // L, step, 0)
    pltpu.sync_copy(res_v, out_h.at[pl.ds(base, bw)])

  run = pl.kernel(
      body,
      out_type=[jax.ShapeDtypeStruct((b,), jnp.float32)],
      mesh=mesh,
      scratch_types=[
          pltpu.VMEM((n, 2), jnp.float32),
          pltpu.VMEM((bw,), jnp.int32),
          pltpu.VMEM((bw,), jnp.int32),
          pltpu.VMEM((bw,), jnp.float32),
      ],
      compiler_params=pltpu.CompilerParams(use_tc_tiling_on_sc=False,
                                           needs_layout_passes=False),
  )
  (out,) = run(uv, a1, a2)
  return out


def kernel(vector_tensor, edge_index_tensor, article1_idx, article2_idx,
           W1_self, W1_neigh, b1, W2_self, W2_neigh, b2, Wc, bc):
  n, d = vector_tensor.shape
  e = edge_index_tensor.shape[1]
  h = W1_self.shape[1]
  b = article1_idx.shape[0]
  ew = e // NW
  ch_c = 100
  ch_r = ew // ch_c
  n_pad = ((n + 8 * NS - 1) // (8 * NS)) * (8 * NS)

  ei3 = edge_index_tensor.astype(jnp.int32).reshape(2, NW, ch_r, ch_c)
  a1 = article1_idx.astype(jnp.int32)
  a2 = article2_idx.astype(jnp.int32)

  z, m1 = pl.pallas_call(
      _mm2_body,
      out_shape=[jax.ShapeDtypeStruct((n_pad, h), jnp.float32),
                 jax.ShapeDtypeStruct((n, h), jnp.float32)],
  )(vector_tensor, W1_self, W1_neigh)

  s1, degp = _edge_pass(ei3, n_pad, table=m1, with_deg=True)

  s2, x1 = _edge_pass(ei3, n_pad, x1_in=(z, s1, degp, b1), with_deg=False)

  uv = pl.pallas_call(
      _uv_body,
      out_shape=jax.ShapeDtypeStruct((n, 2), jnp.float32),
  )(x1, s2, degp, W2_self, W2_neigh, b2.reshape(1, d), Wc,
    bc.reshape(1, 1))

  logits = _pair_gather(uv, a1, a2, b)
  return logits.reshape(b, 1)
